# Initial kernel scaffold; baseline (speedup 1.0000x reference)
import jax
import jax.numpy as jnp
from jax.experimental import pallas as pl


def kernel(hidden_states, gate_w, Wg, Wu, Wd, sWg, sWu, sWd, expert_bias):
    raise NotImplementedError("write your pallas kernel here")


# speedup vs baseline: 2.6464x; 2.6464x over previous
import jax
import jax.numpy as jnp
from jax import lax
from jax.experimental import pallas as pl
from jax.experimental.pallas import tpu as pltpu

_BM = 256


def _moe_body(gate_ref, bias_ref, x_ref, wg_ref, wu_ref, wd_ref,
              swg_ref, swu_ref, swd_ref, out_ref, w_scr, acc_scr):
    e = pl.program_id(0)
    i = pl.program_id(1)
    ne = pl.num_programs(0)

    x = x_ref[...]
    rows = pl.ds(i * _BM, _BM)

    @pl.when(e == 0)
    def _route():
        logits = lax.dot_general(x, gate_ref[...], (((1,), (1,)), ((), ())),
                                 preferred_element_type=jnp.float32)
        scores = jax.nn.sigmoid(logits)
        biased = scores + bias_ref[...]
        ee = lax.broadcasted_iota(jnp.int32, scores.shape, 1)
        m1 = jnp.max(biased, axis=1, keepdims=True)
        i1 = jnp.min(jnp.where(biased == m1, ee, 99), axis=1, keepdims=True)
        b2 = jnp.where(ee == i1, -1e30, biased)
        m2 = jnp.max(b2, axis=1, keepdims=True)
        i2 = jnp.min(jnp.where(b2 == m2, ee, 99), axis=1, keepdims=True)
        s1 = jnp.sum(jnp.where(ee == i1, scores, 0.0), axis=1, keepdims=True)
        s2 = jnp.sum(jnp.where(ee == i2, scores, 0.0), axis=1, keepdims=True)
        denom = s1 + s2 + 1e-20
        w = (jnp.where(ee == i1, s1, 0.0) + jnp.where(ee == i2, s2, 0.0)) / denom
        w_scr[rows, :] = w

    w_row = w_scr[rows, :]
    ee = lax.broadcasted_iota(jnp.int32, w_row.shape, 1)
    wcol = jnp.where(e == 0, 1.0,
                     jnp.sum(jnp.where(ee == e - 1, w_row, 0.0),
                             axis=1, keepdims=True))

    wg = jnp.where(e == 0, swg_ref[...], wg_ref[0])
    wu = jnp.where(e == 0, swu_ref[...], wu_ref[0])
    wd = jnp.where(e == 0, swd_ref[...], wd_ref[0])

    g = lax.dot_general(x, wg, (((1,), (1,)), ((), ())),
                        preferred_element_type=jnp.float32)
    u = lax.dot_general(x, wu, (((1,), (1,)), ((), ())),
                        preferred_element_type=jnp.float32)
    h = g * jax.nn.sigmoid(g) * u
    y = lax.dot_general(h, wd, (((1,), (1,)), ((), ())),
                        preferred_element_type=jnp.float32)

    prev = acc_scr[rows, :]
    acc = jnp.where(e == 0, jnp.zeros_like(prev), prev) + wcol * y
    acc_scr[rows, :] = acc

    @pl.when(e == ne - 1)
    def _flush():
        out_ref[...] = acc


def kernel(hidden_states, gate_w, Wg, Wu, Wd, sWg, sWu, sWd, expert_bias):
    bsz, seq, d = hidden_states.shape
    flat = hidden_states.reshape(-1, d)
    n = flat.shape[0]
    nexp, dff = Wg.shape[0], Wg.shape[1]
    nb = n // _BM
    bias2 = expert_bias.reshape(1, nexp)

    grid = (nexp + 1, nb)
    out = pl.pallas_call(
        _moe_body,
        grid=grid,
        in_specs=[
            pl.BlockSpec((nexp, d), lambda e, i: (0, 0)),
            pl.BlockSpec((1, nexp), lambda e, i: (0, 0)),
            pl.BlockSpec((_BM, d), lambda e, i: (i, 0)),
            pl.BlockSpec((1, dff, d),
                         lambda e, i: (jnp.maximum(e - 1, 0), 0, 0)),
            pl.BlockSpec((1, dff, d),
                         lambda e, i: (jnp.maximum(e - 1, 0), 0, 0)),
            pl.BlockSpec((1, d, dff),
                         lambda e, i: (jnp.maximum(e - 1, 0), 0, 0)),
            pl.BlockSpec((dff, d), lambda e, i: (0, 0)),
            pl.BlockSpec((dff, d), lambda e, i: (0, 0)),
            pl.BlockSpec((d, dff), lambda e, i: (0, 0)),
        ],
        out_specs=pl.BlockSpec((_BM, d), lambda e, i: (i, 0)),
        out_shape=jax.ShapeDtypeStruct((n, d), jnp.float32),
        scratch_shapes=[
            pltpu.VMEM((n, nexp), jnp.float32),
            pltpu.VMEM((n, d), jnp.float32),
        ],
        compiler_params=pltpu.CompilerParams(
            dimension_semantics=("arbitrary", "arbitrary"),
        ),
    )(gate_w, bias2, flat, Wg, Wu, Wd, sWg, sWu, sWd)
    return out.reshape(bsz, seq, d)


# speedup vs baseline: 2.9208x; 1.1037x over previous
import jax
import jax.numpy as jnp
from jax import lax
from jax.experimental import pallas as pl
from jax.experimental.pallas import tpu as pltpu

_BM = 256


def _moe_body(gate_ref, bias_ref, x_ref, wg_ref, wu_ref, wd_ref,
              swg_ref, swu_ref, swd_ref, out_ref, w_scr, acc_scr, x_scr):
    e = pl.program_id(0)
    i = pl.program_id(1)
    ne = pl.num_programs(0)

    rows = pl.ds(i * _BM, _BM)

    @pl.when(e == 0)
    def _stage_x():
        x_scr[rows, :] = x_ref[...]

    x = x_scr[rows, :]

    @pl.when(e == 0)
    def _route():
        logits = lax.dot_general(x, gate_ref[...], (((1,), (1,)), ((), ())),
                                 preferred_element_type=jnp.float32)
        scores = jax.nn.sigmoid(logits)
        biased = scores + bias_ref[...]
        ee = lax.broadcasted_iota(jnp.int32, scores.shape, 1)
        m1 = jnp.max(biased, axis=1, keepdims=True)
        i1 = jnp.min(jnp.where(biased == m1, ee, 99), axis=1, keepdims=True)
        b2 = jnp.where(ee == i1, -1e30, biased)
        m2 = jnp.max(b2, axis=1, keepdims=True)
        i2 = jnp.min(jnp.where(b2 == m2, ee, 99), axis=1, keepdims=True)
        s1 = jnp.sum(jnp.where(ee == i1, scores, 0.0), axis=1, keepdims=True)
        s2 = jnp.sum(jnp.where(ee == i2, scores, 0.0), axis=1, keepdims=True)
        denom = s1 + s2 + 1e-20
        w = (jnp.where(ee == i1, s1, 0.0) + jnp.where(ee == i2, s2, 0.0)) / denom
        w_scr[rows, :] = w

    w_row = w_scr[rows, :]
    ee = lax.broadcasted_iota(jnp.int32, w_row.shape, 1)
    wcol = jnp.where(e == 0, 1.0,
                     jnp.sum(jnp.where(ee == e - 1, w_row, 0.0),
                             axis=1, keepdims=True))

    wg = jnp.where(e == 0, swg_ref[...], wg_ref[0])
    wu = jnp.where(e == 0, swu_ref[...], wu_ref[0])
    wd = jnp.where(e == 0, swd_ref[...], wd_ref[0])

    g = lax.dot_general(x, wg, (((1,), (1,)), ((), ())),
                        preferred_element_type=jnp.float32)
    u = lax.dot_general(x, wu, (((1,), (1,)), ((), ())),
                        preferred_element_type=jnp.float32)
    h = g * jax.nn.sigmoid(g) * u
    y = lax.dot_general(h, wd, (((1,), (1,)), ((), ())),
                        preferred_element_type=jnp.float32)

    prev = acc_scr[rows, :]
    acc = jnp.where(e == 0, jnp.zeros_like(prev), prev) + wcol * y
    acc_scr[rows, :] = acc

    @pl.when(e == ne - 1)
    def _flush():
        out_ref[...] = acc


def kernel(hidden_states, gate_w, Wg, Wu, Wd, sWg, sWu, sWd, expert_bias):
    bsz, seq, d = hidden_states.shape
    flat = hidden_states.reshape(-1, d)
    n = flat.shape[0]
    nexp, dff = Wg.shape[0], Wg.shape[1]
    nb = n // _BM
    bias2 = expert_bias.reshape(1, nexp)

    grid = (nexp + 1, nb)
    out = pl.pallas_call(
        _moe_body,
        grid=grid,
        in_specs=[
            pl.BlockSpec((nexp, d), lambda e, i: (0, 0)),
            pl.BlockSpec((1, nexp), lambda e, i: (0, 0)),
            pl.BlockSpec((_BM, d),
                         lambda e, i: (jnp.where(e == 0, i, nb - 1), 0)),
            pl.BlockSpec((1, dff, d),
                         lambda e, i: (jnp.maximum(e - 1, 0), 0, 0)),
            pl.BlockSpec((1, dff, d),
                         lambda e, i: (jnp.maximum(e - 1, 0), 0, 0)),
            pl.BlockSpec((1, d, dff),
                         lambda e, i: (jnp.maximum(e - 1, 0), 0, 0)),
            pl.BlockSpec((dff, d), lambda e, i: (0, 0)),
            pl.BlockSpec((dff, d), lambda e, i: (0, 0)),
            pl.BlockSpec((d, dff), lambda e, i: (0, 0)),
        ],
        out_specs=pl.BlockSpec(
            (_BM, d), lambda e, i: (jnp.where(e == nexp, i, 0), 0)),
        out_shape=jax.ShapeDtypeStruct((n, d), jnp.float32),
        scratch_shapes=[
            pltpu.VMEM((n, nexp), jnp.float32),
            pltpu.VMEM((n, d), jnp.float32),
            pltpu.VMEM((n, d), jnp.float32),
        ],
        compiler_params=pltpu.CompilerParams(
            dimension_semantics=("arbitrary", "arbitrary"),
        ),
    )(gate_w, bias2, flat, Wg, Wu, Wd, sWg, sWu, sWd)
    return out.reshape(bsz, seq, d)
